# Initial kernel scaffold; baseline (speedup 1.0000x reference)
import jax
import jax.numpy as jnp
from jax.experimental import pallas as pl


def kernel(x, edge_index, edge_attr, Wn, bn_, We, be, conv_W1, conv_b1, conv_W2, conv_b2, bn_gamma, bn_beta, bn_mean, bn_var, mW1, mb1, mW2, mb2, mW3, mb3):
    raise NotImplementedError("write your pallas kernel here")


# speedup vs baseline: 3.6590x; 3.6590x over previous
import functools

import jax
import jax.numpy as jnp
from jax import lax
from jax.experimental import pallas as pl
from jax.experimental.pallas import tpu as pltpu
from jax.experimental.pallas import tpu_sc as plsc

_N = 10000
_E = 320000
_HP = 128
_B = 128
_NBLK = _E // _B
_NW = 32
_NP = 10240
_RPT = _NP // 16
_F32 = jnp.float32


def _sc_agg_body(h_hbm, ea_hbm, src_hbm, dst_hbm, out_hbm,
                 idx_s, idx_d, rows, eabuf, sem_g, sem_e, agg):
    c = lax.axis_index("c")
    s = lax.axis_index("s")
    w = s * 2 + c

    @pl.loop(0, _B)
    def _zero(r):
        for j in range(_HP // 16):
            rows[r, pl.ds(j * 16, 16)] = jnp.zeros((16,), _F32)

    stripe = s * _RPT
    for i in range(5):
        pltpu.sync_copy(rows, agg.at[pl.ds(stripe + i * 128, 128)])
    plsc.subcore_barrier()

    @pl.loop(w, _NBLK, step=_NW)
    def _blk(bi):
        base = bi * _B
        pltpu.sync_copy(src_hbm.at[pl.ds(base, _B)], idx_s)
        cg = pltpu.async_copy(h_hbm.at[idx_s], rows, sem_g)
        ce = pltpu.async_copy(ea_hbm.at[pl.ds(base, _B)], eabuf, sem_e)
        pltpu.sync_copy(dst_hbm.at[pl.ds(base, _B)], idx_d)
        ce.wait()
        cg.wait()

        @pl.loop(0, _B)
        def _row(r):
            for j in range(_HP // 16):
                sl = pl.ds(j * 16, 16)
                rows[r, sl] = jnp.maximum(rows[r, sl] + eabuf[r, sl], 0.0)

        pltpu.sync_copy(rows, agg.at[idx_d], add=True)

    plsc.subcore_barrier()

    for i in range(5):
        off = stripe + i * 128
        pltpu.sync_copy(agg.at[pl.ds(off, 128)],
                        out_hbm.at[pl.ds(c * _NP + off, 128)])


@functools.cache
def _make_sc_agg():
  return pl.kernel(
    _sc_agg_body,
    out_type=jax.ShapeDtypeStruct((2 * _NP, _HP), _F32),
    mesh=plsc.VectorSubcoreMesh(core_axis_name="c", subcore_axis_name="s",
                                num_cores=2, num_subcores=16),
    scratch_types=[
        pltpu.VMEM((_B,), jnp.int32),
        pltpu.VMEM((_B,), jnp.int32),
        pltpu.VMEM((_B, _HP), _F32),
        pltpu.VMEM((_B, _HP), _F32),
        pltpu.SemaphoreType.DMA,
        pltpu.SemaphoreType.DMA,
        pltpu.VMEM_SHARED((_NP, _HP), _F32),
    ],
  )


def _mm_bias_body(a_ref, w_ref, b_ref, o_ref):
    o_ref[...] = jnp.dot(a_ref[...], w_ref[...],
                         preferred_element_type=_F32) + b_ref[...]


def _mm_bias(a, w, b, rb):
    m, k = a.shape
    hc = w.shape[1]
    return pl.pallas_call(
        _mm_bias_body,
        grid=(m // rb,),
        in_specs=[pl.BlockSpec((rb, k), lambda i: (i, 0)),
                  pl.BlockSpec((k, hc), lambda i: (0, 0)),
                  pl.BlockSpec((1, hc), lambda i: (0, 0))],
        out_specs=pl.BlockSpec((rb, hc), lambda i: (i, 0)),
        out_shape=jax.ShapeDtypeStruct((m, hc), _F32),
    )(a, w, b)


def _node_update_body(h_ref, p0_ref, p1_ref, w1_ref, b1_ref, w2_ref, b2_ref,
                      gam_ref, bet_ref, mea_ref, var_ref, o_ref):
    h = h_ref[...]
    z = h + p0_ref[...] + p1_ref[...]
    z = jnp.maximum(jnp.dot(z, w1_ref[...], preferred_element_type=_F32)
                    + b1_ref[...], 0.0)
    z = jnp.dot(z, w2_ref[...], preferred_element_type=_F32) + b2_ref[...]
    scale = gam_ref[...] * lax.rsqrt(var_ref[...] + 1e-5)
    z = (z - mea_ref[...]) * scale + bet_ref[...]
    o_ref[...] = (h + jnp.maximum(z, 0.0)) * 0.5


def _node_update(h, p0, p1, w1, b1, w2, b2, gam, bet, mea, var):
    rb = 1000
    vec = pl.BlockSpec((1, _HP), lambda i: (0, 0))
    return pl.pallas_call(
        _node_update_body,
        grid=(_N // rb,),
        in_specs=[pl.BlockSpec((rb, _HP), lambda i: (i, 0)),
                  pl.BlockSpec((rb, _HP), lambda i: (i, 0)),
                  pl.BlockSpec((rb, _HP), lambda i: (i, 0)),
                  pl.BlockSpec((_HP, _HP), lambda i: (0, 0)), vec,
                  pl.BlockSpec((_HP, _HP), lambda i: (0, 0)), vec,
                  vec, vec, vec, vec],
        out_specs=pl.BlockSpec((rb, _HP), lambda i: (i, 0)),
        out_shape=jax.ShapeDtypeStruct((_N, _HP), _F32),
    )(h, p0, p1, w1, b1, w2, b2, gam, bet, mea, var)


def _head_body(h_ref, w1_ref, b1_ref, w2_ref, b2_ref, w3_ref, b3_ref, o_ref):
    z = jnp.maximum(jnp.dot(h_ref[...], w1_ref[...],
                            preferred_element_type=_F32) + b1_ref[...], 0.0)
    z = jnp.maximum(jnp.dot(z, w2_ref[...],
                            preferred_element_type=_F32) + b2_ref[...], 0.0)
    o_ref[...] = jnp.dot(z, w3_ref[...],
                         preferred_element_type=_F32) + b3_ref[...]


def _head(h, w1, b1, w2, b2, w3, b3):
    rb = 1000
    vec = pl.BlockSpec((1, 128), lambda i: (0, 0))
    return pl.pallas_call(
        _head_body,
        grid=(_N // rb,),
        in_specs=[pl.BlockSpec((rb, _HP), lambda i: (i, 0)),
                  pl.BlockSpec((_HP, 128), lambda i: (0, 0)), vec,
                  pl.BlockSpec((128, 128), lambda i: (0, 0)), vec,
                  pl.BlockSpec((128, 128), lambda i: (0, 0)), vec],
        out_specs=pl.BlockSpec((rb, 128), lambda i: (i, 0)),
        out_shape=jax.ShapeDtypeStruct((_N, 128), _F32),
    )(h, w1, b1, w2, b2, w3, b3)


def _pad_cols(a, w, value=0.0):
    return jnp.pad(a, [(0, 0)] * (a.ndim - 1) + [(0, w - a.shape[-1])],
                   constant_values=value)


def kernel(x, edge_index, edge_attr, Wn, bn_, We, be, conv_W1, conv_b1,
           conv_W2, conv_b2, bn_gamma, bn_beta, bn_mean, bn_var,
           mW1, mb1, mW2, mb2, mW3, mb3):
    src = edge_index[0]
    dst = edge_index[1]

    wn = _pad_cols(Wn, _HP)
    bn = _pad_cols(bn_.reshape(1, -1), _HP)
    we = _pad_cols(We, _HP)
    beb = _pad_cols(be.reshape(1, -1), _HP)
    w1 = _pad_cols(jnp.pad(conv_W1, ((0, 0), (0, _HP - 100), (0, 0))), _HP)
    b1 = _pad_cols(conv_b1, _HP)
    w2 = _pad_cols(jnp.pad(conv_W2, ((0, 0), (0, _HP - 100), (0, 0))), _HP)
    b2 = _pad_cols(conv_b2, _HP)
    gam = _pad_cols(bn_gamma, _HP)
    bet = _pad_cols(bn_beta, _HP)
    mea = _pad_cols(bn_mean, _HP)
    var = _pad_cols(bn_var, _HP, value=1.0)
    hw1 = _pad_cols(jnp.pad(mW1, ((0, _HP - 100), (0, 0))), 128)
    hb1 = _pad_cols(mb1.reshape(1, -1), 128)
    hw2 = _pad_cols(jnp.pad(mW2, ((0, 128 - 50), (0, 0))), 128)
    hb2 = _pad_cols(mb2.reshape(1, -1), 128)
    hw3 = _pad_cols(jnp.pad(mW3, ((0, 128 - 25), (0, 0))), 128)
    hb3 = _pad_cols(mb3.reshape(1, -1), 128)

    h = _mm_bias(x, wn, bn, rb=1000)
    ea = _mm_bias(edge_attr, we, beb, rb=2000)

    for i in range(2):
        parts = _make_sc_agg()(h, ea, src, dst)
        h = _node_update(h, parts[:_N], parts[_NP:_NP + _N],
                         w1[i], b1[i:i + 1], w2[i], b2[i:i + 1],
                         gam[i:i + 1], bet[i:i + 1], mea[i:i + 1],
                         var[i:i + 1])

    out = _head(h, hw1, hb1, hw2, hb2, hw3, hb3)
    return out[:, :2]


# speedup vs baseline: 4.9829x; 1.3618x over previous
import functools

import jax
import jax.numpy as jnp
from jax import lax
from jax.experimental import pallas as pl
from jax.experimental.pallas import tpu as pltpu
from jax.experimental.pallas import tpu_sc as plsc

_N = 10000
_E = 320000
_HP = 128
_B = 80
_KB = 25
_G = 5
_EPT = _E // 32
_NW = 32
_NP = 10112
_RPT = _NP // 16
_F32 = jnp.float32


def _sc_agg_body(h_hbm, ea_hbm, src_hbm, dst_hbm, out_hbm,
                 srcb, dstb, rows0, rows1, ea0, ea1,
                 sg0, sg1, se0, se1, agg):
    c = lax.axis_index("c")
    s = lax.axis_index("s")
    w = s * 2 + c
    rows = (rows0, rows1)
    eab = (ea0, ea1)
    sg = (sg0, sg1)
    se = (se0, se1)

    @pl.loop(0, _B)
    def _zero(r):
        for j in range(_HP // 16):
            rows0[r, pl.ds(j * 16, 16)] = jnp.zeros((16,), _F32)

    stripe = s * _RPT
    for i in range(7):
        pltpu.sync_copy(rows0, agg.at[pl.ds(stripe + i * _B, _B)])
    pltpu.sync_copy(rows0.at[pl.ds(0, _RPT - 7 * _B)],
                    agg.at[pl.ds(stripe + 7 * _B, _RPT - 7 * _B)])
    plsc.subcore_barrier()

    ebase_w = w * _EPT

    @pl.loop(0, _G)
    def _grp(g):
        pltpu.sync_copy(src_hbm.at[w, g], srcb)
        pltpu.sync_copy(dst_hbm.at[w, g], dstb)
        gbase = ebase_w + g * (_KB * _B)

        def issue(k, b):
            pltpu.async_copy(h_hbm.at[srcb.at[k]], rows[b], sg[b])
            pltpu.async_copy(ea_hbm.at[pl.ds(gbase + k * _B, _B)],
                             eab[b], se[b])

        def step(k, b):
            pltpu.make_async_copy(h_hbm.at[srcb.at[k]], rows[b], sg[b]).wait()
            pltpu.make_async_copy(ea_hbm.at[pl.ds(gbase + k * _B, _B)],
                                  eab[b], se[b]).wait()

            @pl.loop(0, _B)
            def _row(r):
                for j in range(_HP // 16):
                    sl = pl.ds(j * 16, 16)
                    rows[b][r, sl] = jnp.maximum(
                        rows[b][r, sl] + eab[b][r, sl], 0.0)

            pltpu.sync_copy(rows[b], agg.at[dstb.at[k]], add=True)

        issue(0, 0)
        issue(1, 1)

        @pl.loop(0, (_KB - 1) // 2)
        def _pair(kk):
            k0 = kk * 2
            step(k0, 0)
            issue(k0 + 2, 0)
            k1 = k0 + 1
            step(k1, 1)

            @pl.when(k1 + 2 < _KB)
            def _():
                issue(k1 + 2, 1)

        step(_KB - 1, 0)

    plsc.subcore_barrier()

    for i in range(4):
        off = stripe + i * 128
        pltpu.sync_copy(agg.at[pl.ds(off, 128)],
                        out_hbm.at[pl.ds(c * _NP + off, 128)])
    off = stripe + 512
    pltpu.sync_copy(agg.at[pl.ds(off, _RPT - 512)],
                    out_hbm.at[pl.ds(c * _NP + off, _RPT - 512)])


@functools.cache
def _make_sc_agg():
  return pl.kernel(
    _sc_agg_body,
    out_type=jax.ShapeDtypeStruct((2 * _NP, _HP), _F32),
    mesh=plsc.VectorSubcoreMesh(core_axis_name="c", subcore_axis_name="s",
                                num_cores=2, num_subcores=16),
    scratch_types=[
        pltpu.VMEM((_KB, _B), jnp.int32),
        pltpu.VMEM((_KB, _B), jnp.int32),
        pltpu.VMEM((_B, _HP), _F32),
        pltpu.VMEM((_B, _HP), _F32),
        pltpu.VMEM((_B, _HP), _F32),
        pltpu.VMEM((_B, _HP), _F32),
        pltpu.SemaphoreType.DMA,
        pltpu.SemaphoreType.DMA,
        pltpu.SemaphoreType.DMA,
        pltpu.SemaphoreType.DMA,
        pltpu.VMEM_SHARED((_NP, _HP), _F32),
    ],
  )


def _mm_bias_body(a_ref, w_ref, b_ref, o_ref):
    o_ref[...] = jnp.dot(a_ref[...], w_ref[...],
                         preferred_element_type=_F32) + b_ref[...]


def _mm_bias(a, w, b, rb):
    m, k = a.shape
    hc = w.shape[1]
    return pl.pallas_call(
        _mm_bias_body,
        grid=(m // rb,),
        in_specs=[pl.BlockSpec((rb, k), lambda i: (i, 0)),
                  pl.BlockSpec((k, hc), lambda i: (0, 0)),
                  pl.BlockSpec((1, hc), lambda i: (0, 0))],
        out_specs=pl.BlockSpec((rb, hc), lambda i: (i, 0)),
        out_shape=jax.ShapeDtypeStruct((m, hc), _F32),
    )(a, w, b)


def _node_update_body(h_ref, p0_ref, p1_ref, w1_ref, b1_ref, w2_ref, b2_ref,
                      gam_ref, bet_ref, mea_ref, var_ref, o_ref):
    h = h_ref[...]
    z = h + p0_ref[...] + p1_ref[...]
    z = jnp.maximum(jnp.dot(z, w1_ref[...], preferred_element_type=_F32)
                    + b1_ref[...], 0.0)
    z = jnp.dot(z, w2_ref[...], preferred_element_type=_F32) + b2_ref[...]
    scale = gam_ref[...] * lax.rsqrt(var_ref[...] + 1e-5)
    z = (z - mea_ref[...]) * scale + bet_ref[...]
    o_ref[...] = (h + jnp.maximum(z, 0.0)) * 0.5


def _node_update(h, p0, p1, w1, b1, w2, b2, gam, bet, mea, var):
    rb = 1000
    vec = pl.BlockSpec((1, _HP), lambda i: (0, 0))
    return pl.pallas_call(
        _node_update_body,
        grid=(_N // rb,),
        in_specs=[pl.BlockSpec((rb, _HP), lambda i: (i, 0)),
                  pl.BlockSpec((rb, _HP), lambda i: (i, 0)),
                  pl.BlockSpec((rb, _HP), lambda i: (i, 0)),
                  pl.BlockSpec((_HP, _HP), lambda i: (0, 0)), vec,
                  pl.BlockSpec((_HP, _HP), lambda i: (0, 0)), vec,
                  vec, vec, vec, vec],
        out_specs=pl.BlockSpec((rb, _HP), lambda i: (i, 0)),
        out_shape=jax.ShapeDtypeStruct((_N, _HP), _F32),
    )(h, p0, p1, w1, b1, w2, b2, gam, bet, mea, var)


def _head_body(h_ref, w1_ref, b1_ref, w2_ref, b2_ref, w3_ref, b3_ref, o_ref):
    z = jnp.maximum(jnp.dot(h_ref[...], w1_ref[...],
                            preferred_element_type=_F32) + b1_ref[...], 0.0)
    z = jnp.maximum(jnp.dot(z, w2_ref[...],
                            preferred_element_type=_F32) + b2_ref[...], 0.0)
    o_ref[...] = jnp.dot(z, w3_ref[...],
                         preferred_element_type=_F32) + b3_ref[...]


def _head(h, w1, b1, w2, b2, w3, b3):
    rb = 1000
    vec = pl.BlockSpec((1, 128), lambda i: (0, 0))
    return pl.pallas_call(
        _head_body,
        grid=(_N // rb,),
        in_specs=[pl.BlockSpec((rb, _HP), lambda i: (i, 0)),
                  pl.BlockSpec((_HP, 128), lambda i: (0, 0)), vec,
                  pl.BlockSpec((128, 128), lambda i: (0, 0)), vec,
                  pl.BlockSpec((128, 128), lambda i: (0, 0)), vec],
        out_specs=pl.BlockSpec((rb, 128), lambda i: (i, 0)),
        out_shape=jax.ShapeDtypeStruct((_N, 128), _F32),
    )(h, w1, b1, w2, b2, w3, b3)


def _pad_cols(a, w, value=0.0):
    return jnp.pad(a, [(0, 0)] * (a.ndim - 1) + [(0, w - a.shape[-1])],
                   constant_values=value)


def kernel(x, edge_index, edge_attr, Wn, bn_, We, be, conv_W1, conv_b1,
           conv_W2, conv_b2, bn_gamma, bn_beta, bn_mean, bn_var,
           mW1, mb1, mW2, mb2, mW3, mb3):
    src = edge_index[0].reshape(_NW, _G, _KB, _B)
    dst = edge_index[1].reshape(_NW, _G, _KB, _B)

    wn = _pad_cols(Wn, _HP)
    bn = _pad_cols(bn_.reshape(1, -1), _HP)
    we = _pad_cols(We, _HP)
    beb = _pad_cols(be.reshape(1, -1), _HP)
    w1 = _pad_cols(jnp.pad(conv_W1, ((0, 0), (0, _HP - 100), (0, 0))), _HP)
    b1 = _pad_cols(conv_b1, _HP)
    w2 = _pad_cols(jnp.pad(conv_W2, ((0, 0), (0, _HP - 100), (0, 0))), _HP)
    b2 = _pad_cols(conv_b2, _HP)
    gam = _pad_cols(bn_gamma, _HP)
    bet = _pad_cols(bn_beta, _HP)
    mea = _pad_cols(bn_mean, _HP)
    var = _pad_cols(bn_var, _HP, value=1.0)
    hw1 = _pad_cols(jnp.pad(mW1, ((0, _HP - 100), (0, 0))), 128)
    hb1 = _pad_cols(mb1.reshape(1, -1), 128)
    hw2 = _pad_cols(jnp.pad(mW2, ((0, 128 - 50), (0, 0))), 128)
    hb2 = _pad_cols(mb2.reshape(1, -1), 128)
    hw3 = _pad_cols(jnp.pad(mW3, ((0, 128 - 25), (0, 0))), 128)
    hb3 = _pad_cols(mb3.reshape(1, -1), 128)

    h = _mm_bias(x, wn, bn, rb=1000)
    ea = _mm_bias(edge_attr, we, beb, rb=2000)

    for i in range(2):
        parts = _make_sc_agg()(h, ea, src, dst)
        h = _node_update(h, parts[:_N], parts[_NP:_NP + _N],
                         w1[i], b1[i:i + 1], w2[i], b2[i:i + 1],
                         gam[i:i + 1], bet[i:i + 1], mea[i:i + 1],
                         var[i:i + 1])

    out = _head(h, hw1, hb1, hw2, hb2, hw3, hb3)
    return out[:, :2]


# speedup vs baseline: 5.0507x; 1.0136x over previous
import functools

import jax
import jax.numpy as jnp
from jax import lax
from jax.experimental import pallas as pl
from jax.experimental.pallas import tpu as pltpu
from jax.experimental.pallas import tpu_sc as plsc

_N = 10000
_E = 320000
_HP = 128
_B = 80
_KB = 25
_G = 5
_EPT = _E // 32
_NW = 32
_NP = 10112
_RPT = _NP // 16
_F32 = jnp.float32


def _sc_agg_body(h_hbm, ea_hbm, src_hbm, dst_hbm, out_hbm,
                 srcb, dstb, rows0, rows1, ea0, ea1,
                 sg0, sg1, se0, se1, agg):
    c = lax.axis_index("c")
    s = lax.axis_index("s")
    w = s * 2 + c
    rows = (rows0, rows1)
    eab = (ea0, ea1)
    sg = (sg0, sg1)
    se = (se0, se1)

    @pl.loop(0, _B)
    def _zero(r):
        for j in range(_HP // 16):
            rows0[r, pl.ds(j * 16, 16)] = jnp.zeros((16,), _F32)

    stripe = s * _RPT
    for i in range(7):
        pltpu.sync_copy(rows0, agg.at[pl.ds(stripe + i * _B, _B)])
    pltpu.sync_copy(rows0.at[pl.ds(0, _RPT - 7 * _B)],
                    agg.at[pl.ds(stripe + 7 * _B, _RPT - 7 * _B)])
    plsc.subcore_barrier()

    ebase_w = w * _EPT

    @pl.loop(0, _G)
    def _grp(g):
        pltpu.sync_copy(src_hbm.at[w, g], srcb)
        pltpu.sync_copy(dst_hbm.at[w, g], dstb)
        gbase = ebase_w + g * (_KB * _B)

        def issue(k, b):
            pltpu.async_copy(h_hbm.at[srcb.at[k]], rows[b], sg[b])
            pltpu.async_copy(ea_hbm.at[pl.ds(gbase + k * _B, _B)],
                             eab[b], se[b])

        def step(k, b):
            pltpu.make_async_copy(h_hbm.at[srcb.at[k]], rows[b], sg[b]).wait()
            pltpu.make_async_copy(ea_hbm.at[pl.ds(gbase + k * _B, _B)],
                                  eab[b], se[b]).wait()

            @pl.loop(0, _B)
            def _row(r):
                for j in range(_HP // 16):
                    sl = pl.ds(j * 16, 16)
                    rows[b][r, sl] = jnp.maximum(
                        rows[b][r, sl] + eab[b][r, sl], 0.0)

            pltpu.sync_copy(rows[b], agg.at[dstb.at[k]], add=True)

        issue(0, 0)
        issue(1, 1)

        @pl.loop(0, (_KB - 1) // 2)
        def _pair(kk):
            k0 = kk * 2
            step(k0, 0)
            issue(k0 + 2, 0)
            k1 = k0 + 1
            step(k1, 1)

            @pl.when(k1 + 2 < _KB)
            def _():
                issue(k1 + 2, 1)

        step(_KB - 1, 0)

    plsc.subcore_barrier()

    for i in range(4):
        off = stripe + i * 128
        pltpu.sync_copy(agg.at[pl.ds(off, 128)],
                        out_hbm.at[pl.ds(c * _NP + off, 128)])
    off = stripe + 512
    pltpu.sync_copy(agg.at[pl.ds(off, _RPT - 512)],
                    out_hbm.at[pl.ds(c * _NP + off, _RPT - 512)])


@functools.cache
def _make_sc_agg():
  return pl.kernel(
    _sc_agg_body,
    out_type=jax.ShapeDtypeStruct((2 * _NP, _HP), _F32),
    mesh=plsc.VectorSubcoreMesh(core_axis_name="c", subcore_axis_name="s",
                                num_cores=2, num_subcores=16),
    scratch_types=[
        pltpu.VMEM((_KB, _B), jnp.int32),
        pltpu.VMEM((_KB, _B), jnp.int32),
        pltpu.VMEM((_B, _HP), _F32),
        pltpu.VMEM((_B, _HP), _F32),
        pltpu.VMEM((_B, _HP), _F32),
        pltpu.VMEM((_B, _HP), _F32),
        pltpu.SemaphoreType.DMA,
        pltpu.SemaphoreType.DMA,
        pltpu.SemaphoreType.DMA,
        pltpu.SemaphoreType.DMA,
        pltpu.VMEM_SHARED((_NP, _HP), _F32),
    ],
  )


def _mm_bias_body(a_ref, w_ref, b_ref, o_ref):
    o_ref[...] = jnp.dot(a_ref[...], w_ref[...],
                         preferred_element_type=_F32) + b_ref[...]


def _mm_bias(a, w, b, rb):
    m, k = a.shape
    hc = w.shape[1]
    return pl.pallas_call(
        _mm_bias_body,
        grid=(m // rb,),
        in_specs=[pl.BlockSpec((rb, k), lambda i: (i, 0)),
                  pl.BlockSpec((k, hc), lambda i: (0, 0)),
                  pl.BlockSpec((1, hc), lambda i: (0, 0))],
        out_specs=pl.BlockSpec((rb, hc), lambda i: (i, 0)),
        out_shape=jax.ShapeDtypeStruct((m, hc), _F32),
    )(a, w, b)


def _node_update_body(h_ref, p0_ref, p1_ref, w1_ref, b1_ref, w2_ref, b2_ref,
                      gam_ref, bet_ref, mea_ref, var_ref, o_ref):
    h = h_ref[...]
    z = h + p0_ref[...] + p1_ref[...]
    z = jnp.maximum(jnp.dot(z, w1_ref[...], preferred_element_type=_F32)
                    + b1_ref[...], 0.0)
    z = jnp.dot(z, w2_ref[...], preferred_element_type=_F32) + b2_ref[...]
    scale = gam_ref[...] * lax.rsqrt(var_ref[...] + 1e-5)
    z = (z - mea_ref[...]) * scale + bet_ref[...]
    o_ref[...] = (h + jnp.maximum(z, 0.0)) * 0.5


def _node_update(h, p0, p1, w1, b1, w2, b2, gam, bet, mea, var):
    rb = 1000
    vec = pl.BlockSpec((1, _HP), lambda i: (0, 0))
    return pl.pallas_call(
        _node_update_body,
        grid=(_N // rb,),
        in_specs=[pl.BlockSpec((rb, _HP), lambda i: (i, 0)),
                  pl.BlockSpec((rb, _HP), lambda i: (i, 0)),
                  pl.BlockSpec((rb, _HP), lambda i: (i, 0)),
                  pl.BlockSpec((_HP, _HP), lambda i: (0, 0)), vec,
                  pl.BlockSpec((_HP, _HP), lambda i: (0, 0)), vec,
                  vec, vec, vec, vec],
        out_specs=pl.BlockSpec((rb, _HP), lambda i: (i, 0)),
        out_shape=jax.ShapeDtypeStruct((_N, _HP), _F32),
    )(h, p0, p1, w1, b1, w2, b2, gam, bet, mea, var)


def _node_head_body(h_ref, p0_ref, p1_ref, w1_ref, b1_ref, w2_ref, b2_ref,
                    gam_ref, bet_ref, mea_ref, var_ref,
                    hw1_ref, hb1_ref, hw2_ref, hb2_ref, hw3_ref, hb3_ref,
                    o_ref):
    h = h_ref[...]
    z = h + p0_ref[...] + p1_ref[...]
    z = jnp.maximum(jnp.dot(z, w1_ref[...], preferred_element_type=_F32)
                    + b1_ref[...], 0.0)
    z = jnp.dot(z, w2_ref[...], preferred_element_type=_F32) + b2_ref[...]
    scale = gam_ref[...] * lax.rsqrt(var_ref[...] + 1e-5)
    z = (z - mea_ref[...]) * scale + bet_ref[...]
    hn = (h + jnp.maximum(z, 0.0)) * 0.5
    z = jnp.maximum(jnp.dot(hn, hw1_ref[...],
                            preferred_element_type=_F32) + hb1_ref[...], 0.0)
    z = jnp.maximum(jnp.dot(z, hw2_ref[...],
                            preferred_element_type=_F32) + hb2_ref[...], 0.0)
    o_ref[...] = jnp.dot(z, hw3_ref[...],
                         preferred_element_type=_F32) + hb3_ref[...]


def _node_head(h, p0, p1, w1, b1, w2, b2, gam, bet, mea, var,
               hw1, hb1, hw2, hb2, hw3, hb3):
    rb = 1000
    vec = pl.BlockSpec((1, _HP), lambda i: (0, 0))
    vec8 = pl.BlockSpec((1, 128), lambda i: (0, 0))
    mat = pl.BlockSpec((_HP, _HP), lambda i: (0, 0))
    mat8 = pl.BlockSpec((128, 128), lambda i: (0, 0))
    return pl.pallas_call(
        _node_head_body,
        grid=(_N // rb,),
        in_specs=[pl.BlockSpec((rb, _HP), lambda i: (i, 0)),
                  pl.BlockSpec((rb, _HP), lambda i: (i, 0)),
                  pl.BlockSpec((rb, _HP), lambda i: (i, 0)),
                  mat, vec, mat, vec, vec, vec, vec, vec,
                  pl.BlockSpec((_HP, 128), lambda i: (0, 0)), vec8,
                  mat8, vec8, mat8, vec8],
        out_specs=pl.BlockSpec((rb, 128), lambda i: (i, 0)),
        out_shape=jax.ShapeDtypeStruct((_N, 128), _F32),
    )(h, p0, p1, w1, b1, w2, b2, gam, bet, mea, var,
      hw1, hb1, hw2, hb2, hw3, hb3)


def _head_body(h_ref, w1_ref, b1_ref, w2_ref, b2_ref, w3_ref, b3_ref, o_ref):
    z = jnp.maximum(jnp.dot(h_ref[...], w1_ref[...],
                            preferred_element_type=_F32) + b1_ref[...], 0.0)
    z = jnp.maximum(jnp.dot(z, w2_ref[...],
                            preferred_element_type=_F32) + b2_ref[...], 0.0)
    o_ref[...] = jnp.dot(z, w3_ref[...],
                         preferred_element_type=_F32) + b3_ref[...]


def _head(h, w1, b1, w2, b2, w3, b3):
    rb = 1000
    vec = pl.BlockSpec((1, 128), lambda i: (0, 0))
    return pl.pallas_call(
        _head_body,
        grid=(_N // rb,),
        in_specs=[pl.BlockSpec((rb, _HP), lambda i: (i, 0)),
                  pl.BlockSpec((_HP, 128), lambda i: (0, 0)), vec,
                  pl.BlockSpec((128, 128), lambda i: (0, 0)), vec,
                  pl.BlockSpec((128, 128), lambda i: (0, 0)), vec],
        out_specs=pl.BlockSpec((rb, 128), lambda i: (i, 0)),
        out_shape=jax.ShapeDtypeStruct((_N, 128), _F32),
    )(h, w1, b1, w2, b2, w3, b3)


def _pad_cols(a, w, value=0.0):
    return jnp.pad(a, [(0, 0)] * (a.ndim - 1) + [(0, w - a.shape[-1])],
                   constant_values=value)


def kernel(x, edge_index, edge_attr, Wn, bn_, We, be, conv_W1, conv_b1,
           conv_W2, conv_b2, bn_gamma, bn_beta, bn_mean, bn_var,
           mW1, mb1, mW2, mb2, mW3, mb3):
    src = edge_index[0].reshape(_NW, _G, _KB, _B)
    dst = edge_index[1].reshape(_NW, _G, _KB, _B)

    wn = _pad_cols(Wn, _HP)
    bn = _pad_cols(bn_.reshape(1, -1), _HP)
    we = _pad_cols(We, _HP)
    beb = _pad_cols(be.reshape(1, -1), _HP)
    w1 = _pad_cols(jnp.pad(conv_W1, ((0, 0), (0, _HP - 100), (0, 0))), _HP)
    b1 = _pad_cols(conv_b1, _HP)
    w2 = _pad_cols(jnp.pad(conv_W2, ((0, 0), (0, _HP - 100), (0, 0))), _HP)
    b2 = _pad_cols(conv_b2, _HP)
    gam = _pad_cols(bn_gamma, _HP)
    bet = _pad_cols(bn_beta, _HP)
    mea = _pad_cols(bn_mean, _HP)
    var = _pad_cols(bn_var, _HP, value=1.0)
    hw1 = _pad_cols(jnp.pad(mW1, ((0, _HP - 100), (0, 0))), 128)
    hb1 = _pad_cols(mb1.reshape(1, -1), 128)
    hw2 = _pad_cols(jnp.pad(mW2, ((0, 128 - 50), (0, 0))), 128)
    hb2 = _pad_cols(mb2.reshape(1, -1), 128)
    hw3 = _pad_cols(jnp.pad(mW3, ((0, 128 - 25), (0, 0))), 128)
    hb3 = _pad_cols(mb3.reshape(1, -1), 128)

    h = _mm_bias(x, wn, bn, rb=1000)
    ea = _mm_bias(edge_attr, we, beb, rb=2000)

    parts = _make_sc_agg()(h, ea, src, dst)
    h = _node_update(h, parts[:_N], parts[_NP:_NP + _N],
                     w1[0], b1[0:1], w2[0], b2[0:1],
                     gam[0:1], bet[0:1], mea[0:1], var[0:1])
    parts = _make_sc_agg()(h, ea, src, dst)
    out = _node_head(h, parts[:_N], parts[_NP:_NP + _N],
                     w1[1], b1[1:2], w2[1], b2[1:2],
                     gam[1:2], bet[1:2], mea[1:2], var[1:2],
                     hw1, hb1, hw2, hb2, hw3, hb3)
    return out[:, :2]
